# baseline (speedup 1.0000x reference)
import functools

import jax
import jax.numpy as jnp
from jax import lax
from jax.experimental import pallas as pl
from jax.experimental.pallas import tpu as pltpu
from jax.experimental.pallas import tpu_sc as plsc

_V = 1000
_H = 128
_NC = 2
_NS = 16
_NW = _NC * _NS


def _table_body(e_ref, w_ref, b_ref, t_ref):
    t_ref[...] = (
        jnp.dot(e_ref[...], w_ref[...].T, preferred_element_type=jnp.float32)
        + b_ref[...]
    )


def _make_table(embed, W, b2d):
    return pl.pallas_call(
        _table_body,
        out_shape=jax.ShapeDtypeStruct((_V, _V), jnp.float32),
    )(embed, W, b2d)


def _make_gather(bsz, seq, half, half_pad):
    slabs_per_w = bsz // _NW
    n_chunks = 2 * slabs_per_w
    mesh = plsc.VectorSubcoreMesh(core_axis_name="c", subcore_axis_name="s")

    @functools.partial(
        pl.kernel,
        mesh=mesh,
        compiler_params=pltpu.CompilerParams(use_tc_tiling_on_sc=False),
        out_type=jax.ShapeDtypeStruct((bsz, seq, _V), jnp.float32),
        scratch_types=[
            pltpu.VMEM_SHARED((_V, _V), jnp.float32),
            pltpu.VMEM((n_chunks, half_pad), jnp.int32),
            pltpu.VMEM((half_pad, _V), jnp.float32),
            pltpu.VMEM((half_pad, _V), jnp.float32),
            pltpu.SemaphoreType.DMA,
            pltpu.SemaphoreType.DMA,
            pltpu.SemaphoreType.DMA,
            pltpu.SemaphoreType.DMA,
        ],
    )
    def gather(table_hbm, idx_hbm, out_hbm, table_sh, idx_v,
               r0, r1, gs0, gs1, ws0, ws1):
        sid = lax.axis_index("s")
        wid = sid * _NC + lax.axis_index("c")
        base = wid * slabs_per_w

        @pl.when(sid == 0)
        def _():
            pltpu.sync_copy(table_hbm, table_sh)

        pltpu.sync_copy(idx_hbm.at[pl.ds(base * 2, n_chunks)], idx_v)
        plsc.subcore_barrier()

        bufs = ((r0, gs0, ws0), (r1, gs1, ws1))

        def start_gather(ci, r, gs):
            pltpu.async_copy(table_sh.at[idx_v.at[ci]], r, gs)

        def out_slice(ci):
            slab = base + lax.div(ci, 2)
            off = lax.rem(ci, 2) * half
            return out_hbm.at[slab, pl.ds(off, half)]

        for b in range(2):
            start_gather(b, bufs[b][0], bufs[b][1])

        def body(i, carry):
            g = i * 2
            for b in range(2):
                ci = g + b
                r, gs, ws = bufs[b]
                pltpu.make_async_copy(table_sh.at[idx_v.at[ci]], r, gs).wait()
                pltpu.async_copy(r.at[pl.ds(0, half)], out_slice(ci), ws)
            for b in range(2):
                ci = g + b
                r, gs, ws = bufs[b]

                @pl.when(ci + 2 < n_chunks)
                def _():
                    pltpu.make_async_copy(
                        r.at[pl.ds(0, half)], out_slice(ci), ws
                    ).wait()
                    start_gather(ci + 2, r, gs)

            return carry

        lax.fori_loop(0, n_chunks // 2, body, 0)

        for b in range(2):
            r, gs, ws = bufs[b]
            pltpu.make_async_copy(
                r.at[pl.ds(0, half)], out_slice(b), ws
            ).wait()

    return gather


def kernel(input_ids, embed, W, b):
    bsz, seq = input_ids.shape
    half = seq // 2
    half_pad = ((half + 7) // 8) * 8
    table = _make_table(embed, W, b.reshape(1, _V))
    ids = jnp.pad(
        input_ids.astype(jnp.int32).reshape(bsz * 2, half),
        ((0, 0), (0, half_pad - half)),
    )
    return _make_gather(bsz, seq, half, half_pad)(table, ids)


# speedup vs baseline: 2.3866x; 2.3866x over previous
import functools

import jax
import jax.numpy as jnp
from jax import lax
from jax.experimental import pallas as pl
from jax.experimental.pallas import tpu as pltpu
from jax.experimental.pallas import tpu_sc as plsc

_V = 1000
_H = 128
_NC = 2
_NS = 16
_NW = _NC * _NS
_LB = 128
_NJ = 8
_TAIL = _V - (_NJ - 1) * _LB


def _tableR_body(e_ref, w_ref, b_ref, t_ref):
    t_ref[...] = (
        jnp.dot(e_ref[...], w_ref[...].T, preferred_element_type=jnp.float32)
        + b_ref[...]
    )


def _make_tableR(embed, Wp, bp2d):
    return pl.pallas_call(
        _tableR_body,
        grid=(_NJ,),
        in_specs=[
            pl.BlockSpec((_V, _H), lambda j: (0, 0)),
            pl.BlockSpec((_LB, _H), lambda j: (j, 0)),
            pl.BlockSpec((1, _LB), lambda j: (0, j)),
        ],
        out_specs=pl.BlockSpec((_V, _LB), lambda j: (j, 0)),
        out_shape=jax.ShapeDtypeStruct((_NJ * _V, _LB), jnp.float32),
    )(embed, Wp, bp2d)


def _make_gather(bsz, seq, seq_pad):
    slabs_per_w = bsz // _NW
    n_chunks = slabs_per_w * _NJ
    mesh = plsc.VectorSubcoreMesh(core_axis_name="c", subcore_axis_name="s")
    nbuf = 4

    @functools.partial(
        pl.kernel,
        mesh=mesh,
        out_type=(
            jax.ShapeDtypeStruct((bsz, seq, _V), jnp.float32),
            jax.ShapeDtypeStruct((bsz, seq, _LB), jnp.float32),
        ),
        scratch_types=[
            pltpu.VMEM_SHARED((_NJ * _V, _LB), jnp.float32),
            pltpu.VMEM((slabs_per_w, _NJ, seq_pad), jnp.int32),
        ]
        + [pltpu.VMEM((seq, _LB), jnp.float32) for _ in range(nbuf)]
        + [pltpu.SemaphoreType.DMA] * (2 * nbuf),
    )
    def gather(table_hbm, idx_hbm, out_hbm, tail_hbm, table_sh, idx_v, *bufsem):
        bufs = bufsem[:nbuf]
        gsems = bufsem[nbuf:2 * nbuf]
        wsems = bufsem[2 * nbuf:]
        sid = lax.axis_index("s")
        wid = sid * _NC + lax.axis_index("c")
        base = wid * slabs_per_w

        @pl.when(sid == 0)
        def _():
            pltpu.sync_copy(table_hbm, table_sh)

        pltpu.sync_copy(idx_hbm.at[pl.ds(base, slabs_per_w)], idx_v)
        plsc.subcore_barrier()

        def start_gather(s, j, b):
            pltpu.async_copy(
                table_sh.at[idx_v.at[s, j, pl.ds(0, seq)]], bufs[b], gsems[b]
            )

        def wait_gather(s, j, b):
            pltpu.make_async_copy(
                table_sh.at[idx_v.at[s, j, pl.ds(0, seq)]], bufs[b], gsems[b]
            ).wait()

        def out_block(s, j):
            if j == _NJ - 1:
                return tail_hbm.at[base + s]
            return out_hbm.at[base + s, :, pl.ds(j * _LB, _LB)]

        def start_write(s, j, b):
            pltpu.async_copy(bufs[b], out_block(s, j), wsems[b])

        def wait_write(s, j, b):
            pltpu.make_async_copy(bufs[b], out_block(s, j), wsems[b]).wait()

        for j in range(nbuf):
            start_gather(0, j, j)

        def body(s, carry):
            for j in range(_NJ):
                b = j % nbuf
                wait_gather(s, j, b)
                start_write(s, j, b)
                nj = (j + nbuf) % _NJ
                ns = s + (1 if j + nbuf >= _NJ else 0)

                @pl.when(ns < slabs_per_w)
                def _():
                    wait_write(s, j, b)
                    start_gather(ns, nj, b)

            return carry

        lax.fori_loop(0, slabs_per_w, body, 0)

        last = slabs_per_w - 1
        for j in range(nbuf, _NJ):
            wait_write(last, j, j % nbuf)

    return gather


def kernel(input_ids, embed, W, b):
    bsz, seq = input_ids.shape
    seq_pad = ((seq + 7) // 8) * 8
    Wp = jnp.pad(W, ((0, _NJ * _LB - _V), (0, 0)))
    bp = jnp.pad(b, (0, _NJ * _LB - _V))
    tableR = _make_tableR(embed, Wp, bp.reshape(1, _NJ * _LB))
    ids = jnp.pad(input_ids.astype(jnp.int32), ((0, 0), (0, seq_pad - seq)))
    ids8 = ids[:, None, :] + (jnp.arange(_NJ, dtype=jnp.int32) * _V)[None, :, None]
    out_main, out_tail = _make_gather(bsz, seq, seq_pad)(tableR, ids8)
    return lax.dynamic_update_slice(
        out_main, out_tail[:, :, :_TAIL], (0, 0, (_NJ - 1) * _LB)
    )
